# baseline (speedup 1.0000x reference)
import functools

import jax
import jax.numpy as jnp
from jax import lax
from jax.experimental import pallas as pl
from jax.experimental.pallas import tpu as pltpu
from jax.experimental.pallas import tpu_sc as plsc

N = 10000
E = 320000
D = 128

NC = 2
NS = 16
L = 16
NW = NC * NS

IDXC = 128
CH = 80
PH = 4
PCH = CH // PH
NBUF = 2
ROUNDS = PCH // NBUF
EPT = CH * IDXC
EPAD = NW * EPT
NPAD = 10240
RPT = NPAD // NS

_mesh = plsc.VectorSubcoreMesh(core_axis_name="c", subcore_axis_name="s")


def _deg_body(dst3, ew3, out, dstbuf, ewbuf, stage, degsh, sem):
    c = lax.axis_index("c")
    s = lax.axis_index("s")
    w = c * NS + s

    @pl.loop(0, RPT // L)
    def _zero(i):
        stage[pl.ds(i * L, L)] = jnp.zeros((L,), jnp.float32)

    pltpu.sync_copy(stage, degsh.at[pl.ds(s * RPT, RPT)])
    plsc.subcore_barrier()

    pltpu.sync_copy(dst3.at[w], dstbuf)
    pltpu.sync_copy(ew3.at[w], ewbuf)

    @pl.loop(0, CH)
    def _scat(j):
        pltpu.make_async_copy(
            ewbuf.at[j], degsh.at[dstbuf.at[j]], sem).start(add=True)

    @pl.loop(0, CH)
    def _drain(j):
        pltpu.make_async_copy(
            ewbuf.at[j], degsh.at[dstbuf.at[j]], sem).wait()

    plsc.subcore_barrier()
    pltpu.sync_copy(degsh.at[pl.ds(s * RPT, RPT)], stage)
    pltpu.sync_copy(stage, out.at[c, pl.ds(s * RPT, RPT)])


_deg_call = pl.kernel(
    _deg_body,
    out_type=jax.ShapeDtypeStruct((NC, NPAD), jnp.float32),
    mesh=_mesh,
    scratch_types=[
        pltpu.VMEM((CH, IDXC), jnp.int32),
        pltpu.VMEM((CH, IDXC), jnp.float32),
        pltpu.VMEM((RPT,), jnp.float32),
        pltpu.VMEM_SHARED((NPAD,), jnp.float32),
        pltpu.SemaphoreType.DMA,
    ],
)


def _msg_body(xws, src4, dst4, ew4, out,
              srcbuf, dstbuf, ewbuf,
              r0, r1,
              ssh,
              g0, g1, s0, s1):
    c = lax.axis_index("c")
    s = lax.axis_index("s")
    w = c * NS + s
    rows = [r0, r1]
    gsem = [g0, g1]
    ssem = [s0, s1]

    @pl.loop(0, IDXC)
    def _zrow(e):
        for k in range(D // L):
            r0[e, pl.ds(k * L, L)] = jnp.zeros((L,), jnp.float32)

    for t in range(RPT // IDXC):
        pltpu.sync_copy(r0, ssh.at[pl.ds(s * RPT + t * IDXC, IDXC)])
    plsc.subcore_barrier()

    def _gather_start(j, b):
        pltpu.make_async_copy(
            xws.at[srcbuf.at[j]], rows[b], gsem[b]).start()

    def _gather_wait(j, b):
        pltpu.make_async_copy(
            xws.at[srcbuf.at[j]], rows[b], gsem[b]).wait()

    def _scatter_start(j, b):
        pltpu.make_async_copy(
            rows[b], ssh.at[dstbuf.at[j]], ssem[b]).start(add=True)

    def _scatter_wait(j, b):
        pltpu.make_async_copy(
            rows[b], ssh.at[dstbuf.at[j]], ssem[b]).wait()

    def _scale(j, b):
        @pl.loop(0, IDXC // L)
        def _grp(g):
            ewv = ewbuf[j, pl.ds(g * L, L)]
            for t in range(L):
                wvec = jnp.full((L,), ewv[t], jnp.float32)
                e = g * L + t
                for k in range(D // L):
                    sl = pl.ds(k * L, L)
                    rows[b][e, sl] = rows[b][e, sl] * wvec

    for p in range(PH):
        pltpu.sync_copy(src4.at[w, p], srcbuf)
        pltpu.sync_copy(dst4.at[w, p], dstbuf)
        pltpu.sync_copy(ew4.at[w, p], ewbuf)

        _gather_start(0, 0)
        _gather_start(1, 1)

        @pl.loop(0, ROUNDS - 1)
        def _round(i):
            j0 = i * NBUF
            j1 = j0 + 1
            _gather_wait(j0, 0)
            _scale(j0, 0)
            _scatter_start(j0, 0)
            _gather_wait(j1, 1)
            _scale(j1, 1)
            _scatter_wait(j0, 0)
            _gather_start(j0 + NBUF, 0)
            _scatter_start(j1, 1)
            _scatter_wait(j1, 1)
            _gather_start(j1 + NBUF, 1)

        j0 = PCH - 2
        j1 = PCH - 1
        _gather_wait(j0, 0)
        _scale(j0, 0)
        _scatter_start(j0, 0)
        _gather_wait(j1, 1)
        _scale(j1, 1)
        _scatter_start(j1, 1)
        _scatter_wait(j0, 0)
        _scatter_wait(j1, 1)

    plsc.subcore_barrier()
    for t in range(RPT // IDXC):
        rr = s * RPT + t * IDXC
        pltpu.sync_copy(ssh.at[pl.ds(rr, IDXC)], r0)
        pltpu.sync_copy(r0, out.at[c, pl.ds(rr, IDXC)])
    plsc.subcore_barrier()


_msg_call = pl.kernel(
    _msg_body,
    out_type=jax.ShapeDtypeStruct((NC, NPAD, D), jnp.float32),
    mesh=_mesh,
    scratch_types=(
        [pltpu.VMEM((PCH, IDXC), jnp.int32),
         pltpu.VMEM((PCH, IDXC), jnp.int32),
         pltpu.VMEM((PCH, IDXC), jnp.float32)]
        + [pltpu.VMEM((IDXC, D), jnp.float32) for _ in range(NBUF)]
        + [pltpu.VMEM_SHARED((NPAD, D), jnp.float32)]
        + [pltpu.SemaphoreType.DMA for _ in range(2 * NBUF)]
    ),
)


def _lstm_body(w_ref, wih_ref, b_ref, out_ref):
    gates = lax.dot_general(
        w_ref[...], wih_ref[...], (((1,), (1,)), ((), ())),
        preferred_element_type=jnp.float32,
    ) + b_ref[...]
    i_g = gates[:, :D]
    g_g = gates[:, 2 * D:3 * D]
    o_g = gates[:, 3 * D:]
    cc = jax.nn.sigmoid(i_g) * jnp.tanh(g_g)
    out_ref[...] = jax.nn.sigmoid(o_g) * jnp.tanh(cc)


_lstm_call = pl.pallas_call(
    _lstm_body,
    out_shape=jax.ShapeDtypeStruct((D, D), jnp.float32),
)

ROWB = 1024
NBLK = NPAD // ROWB


def _xws_body(x_ref, wn_ref, d0_ref, d1_ref, xws_ref, dis_ref):
    xw = jnp.dot(x_ref[...], wn_ref[...], preferred_element_type=jnp.float32)
    dis = lax.rsqrt(d0_ref[...] + d1_ref[...] + 1.0)
    dis_ref[...] = dis
    xws_ref[...] = dis * xw


_xws_call = pl.pallas_call(
    _xws_body,
    grid=(NBLK,),
    in_specs=[
        pl.BlockSpec((ROWB, D), lambda i: (i, 0)),
        pl.BlockSpec((D, D), lambda i: (0, 0)),
        pl.BlockSpec((ROWB, 1), lambda i: (i, 0)),
        pl.BlockSpec((ROWB, 1), lambda i: (i, 0)),
    ],
    out_specs=[
        pl.BlockSpec((ROWB, D), lambda i: (i, 0)),
        pl.BlockSpec((ROWB, 1), lambda i: (i, 0)),
    ],
    out_shape=[
        jax.ShapeDtypeStruct((NPAD, D), jnp.float32),
        jax.ShapeDtypeStruct((NPAD, 1), jnp.float32),
    ],
)


def _final_body(s0_ref, s1_ref, xws_ref, dis_ref, lw_ref, lb_ref, y_ref):
    h = jnp.maximum(
        dis_ref[...] * (s0_ref[...] + s1_ref[...] + xws_ref[...]), 0.0)
    y = jnp.dot(h, lw_ref[...], preferred_element_type=jnp.float32)
    y_ref[...] = y + lb_ref[0, 0]


_final_call = pl.pallas_call(
    _final_body,
    grid=(NBLK,),
    in_specs=[
        pl.BlockSpec((ROWB, D), lambda i: (i, 0)),
        pl.BlockSpec((ROWB, D), lambda i: (i, 0)),
        pl.BlockSpec((ROWB, D), lambda i: (i, 0)),
        pl.BlockSpec((ROWB, 1), lambda i: (i, 0)),
        pl.BlockSpec((D, 8), lambda i: (0, 0)),
        pl.BlockSpec((1, 1), lambda i: (0, 0)),
    ],
    out_specs=pl.BlockSpec((ROWB, 8), lambda i: (i, 0)),
    out_shape=jax.ShapeDtypeStruct((NPAD, 8), jnp.float32),
)


def kernel(x, edge_index, edge_weight, W, W_ih, W_hh, b_ih, b_hh, lin_w, lin_b):
    src = edge_index[0]
    dst = edge_index[1]
    pad = EPAD - E
    zi = jnp.zeros((pad,), jnp.int32)
    srcp = jnp.concatenate([src, zi])
    dstp = jnp.concatenate([dst, zi])
    ewp = jnp.concatenate([edge_weight, jnp.zeros((pad,), jnp.float32)])

    dst3 = dstp.reshape(NW, CH, IDXC)
    ew3 = ewp.reshape(NW, CH, IDXC)
    src4 = srcp.reshape(NW, PH, PCH, IDXC)
    dst4 = dstp.reshape(NW, PH, PCH, IDXC)
    ew4 = ewp.reshape(NW, PH, PCH, IDXC)

    deg = _deg_call(dst3, ew3)
    d0 = deg[0].reshape(NPAD, 1)
    d1 = deg[1].reshape(NPAD, 1)

    w_new = _lstm_call(W, W_ih, (b_ih + b_hh).reshape(1, 4 * D))

    xp = jnp.pad(x, ((0, NPAD - N), (0, 0)))
    xws, dis = _xws_call(xp, w_new, d0, d1)

    s_part = _msg_call(xws, src4, dst4, ew4)

    lwp = jnp.pad(lin_w.T, ((0, 0), (0, 7)))
    yp = _final_call(s_part[0], s_part[1], xws, dis, lwp, lin_b.reshape(1, 1))
    return yp[:N, 0]


# speedup vs baseline: 2.6377x; 2.6377x over previous
import functools

import jax
import jax.numpy as jnp
from jax import lax
from jax.experimental import pallas as pl
from jax.experimental.pallas import tpu as pltpu
from jax.experimental.pallas import tpu_sc as plsc

N = 10000
E = 320000
D = 128

NC = 2
NS = 16
L = 16
NW = NC * NS

IDXC = 128
CH = 80
PH = 4
PCH = CH // PH
NBUF = 2
ROUNDS = PCH // NBUF
EPT = CH * IDXC
EPAD = NW * EPT
NPAD = 10240
RPT = NPAD // NS

_mesh = plsc.VectorSubcoreMesh(core_axis_name="c", subcore_axis_name="s")


def _deg_body(dst3, ew3, out, dstbuf, ewbuf, stage, degsh, sem):
    c = lax.axis_index("c")
    s = lax.axis_index("s")
    w = c * NS + s

    @pl.loop(0, RPT // L)
    def _zero(i):
        stage[pl.ds(i * L, L)] = jnp.zeros((L,), jnp.float32)

    pltpu.sync_copy(stage, degsh.at[pl.ds(s * RPT, RPT)])
    plsc.subcore_barrier()

    pltpu.sync_copy(dst3.at[w], dstbuf)
    pltpu.sync_copy(ew3.at[w], ewbuf)

    @pl.loop(0, CH)
    def _scat(j):
        pltpu.make_async_copy(
            ewbuf.at[j], degsh.at[dstbuf.at[j]], sem).start(add=True)

    @pl.loop(0, CH)
    def _drain(j):
        pltpu.make_async_copy(
            ewbuf.at[j], degsh.at[dstbuf.at[j]], sem).wait()

    plsc.subcore_barrier()
    pltpu.sync_copy(degsh.at[pl.ds(s * RPT, RPT)], stage)
    pltpu.sync_copy(stage, out.at[c, pl.ds(s * RPT, RPT)])


_deg_call = pl.kernel(
    _deg_body,
    out_type=jax.ShapeDtypeStruct((NC, NPAD), jnp.float32),
    mesh=_mesh,
    scratch_types=[
        pltpu.VMEM((CH, IDXC), jnp.int32),
        pltpu.VMEM((CH, IDXC), jnp.float32),
        pltpu.VMEM((RPT,), jnp.float32),
        pltpu.VMEM_SHARED((NPAD,), jnp.float32),
        pltpu.SemaphoreType.DMA,
    ],
)


def _msg_body(xws, src4, dst4, ew4, out,
              srcbuf, dstbuf, ewbuf,
              r0, r1,
              ssh,
              g0, g1, s0, s1):
    c = lax.axis_index("c")
    s = lax.axis_index("s")
    w = c * NS + s
    rows = [r0, r1]
    gsem = [g0, g1]
    ssem = [s0, s1]

    @pl.loop(0, IDXC)
    def _zrow(e):
        for k in range(D // L):
            r0[e, pl.ds(k * L, L)] = jnp.zeros((L,), jnp.float32)

    for t in range(RPT // IDXC):
        pltpu.sync_copy(r0, ssh.at[pl.ds(s * RPT + t * IDXC, IDXC)])
    plsc.subcore_barrier()

    def _gather_start(j, b):
        pltpu.make_async_copy(
            xws.at[srcbuf.at[j]], rows[b], gsem[b]).start()

    def _gather_wait(j, b):
        pltpu.make_async_copy(
            xws.at[srcbuf.at[j]], rows[b], gsem[b]).wait()

    def _scatter_start(j, b):
        pltpu.make_async_copy(
            rows[b], ssh.at[dstbuf.at[j]], ssem[b]).start(add=True)

    def _scatter_wait(j, b):
        pltpu.make_async_copy(
            rows[b], ssh.at[dstbuf.at[j]], ssem[b]).wait()

    def _scale(j, b):
        @pl.loop(0, IDXC // L)
        def _grp(g):
            ewv = ewbuf[j, pl.ds(g * L, L)]
            for t in range(L):
                wvec = jnp.full((L,), ewv[t], jnp.float32)
                e = g * L + t
                for k in range(D // L):
                    sl = pl.ds(k * L, L)
                    rows[b][e, sl] = rows[b][e, sl] * wvec

    for p in range(PH):
        pltpu.sync_copy(src4.at[w, p], srcbuf)
        pltpu.sync_copy(dst4.at[w, p], dstbuf)
        pltpu.sync_copy(ew4.at[w, p], ewbuf)

        _gather_start(0, 0)
        _gather_start(1, 1)

        @pl.loop(0, ROUNDS - 1)
        def _round(i):
            j0 = i * NBUF
            j1 = j0 + 1
            _gather_wait(j0, 0)
            _scale(j0, 0)
            _scatter_start(j0, 0)
            _gather_wait(j1, 1)
            _scale(j1, 1)
            _scatter_wait(j0, 0)
            _gather_start(j0 + NBUF, 0)
            _scatter_start(j1, 1)
            _scatter_wait(j1, 1)
            _gather_start(j1 + NBUF, 1)

        j0 = PCH - 2
        j1 = PCH - 1
        _gather_wait(j0, 0)
        _scale(j0, 0)
        _scatter_start(j0, 0)
        _gather_wait(j1, 1)
        _scale(j1, 1)
        _scatter_start(j1, 1)
        _scatter_wait(j0, 0)
        _scatter_wait(j1, 1)

    plsc.subcore_barrier()
    for t in range(RPT // IDXC):
        rr = s * RPT + t * IDXC
        pltpu.sync_copy(ssh.at[pl.ds(rr, IDXC)], r0)
        pltpu.sync_copy(r0, out.at[c, pl.ds(rr, IDXC)])
    plsc.subcore_barrier()


_msg_call = pl.kernel(
    _msg_body,
    out_type=jax.ShapeDtypeStruct((NC, NPAD, D), jnp.float32),
    mesh=_mesh,
    scratch_types=(
        [pltpu.VMEM((PCH, IDXC), jnp.int32),
         pltpu.VMEM((PCH, IDXC), jnp.int32),
         pltpu.VMEM((PCH, IDXC), jnp.float32)]
        + [pltpu.VMEM((IDXC, D), jnp.float32) for _ in range(NBUF)]
        + [pltpu.VMEM_SHARED((NPAD, D), jnp.float32)]
        + [pltpu.SemaphoreType.DMA for _ in range(2 * NBUF)]
    ),
)


def _lstm_body(w_ref, wih_ref, b_ref, out_ref):
    gates = lax.dot_general(
        w_ref[...], wih_ref[...], (((1,), (1,)), ((), ())),
        preferred_element_type=jnp.float32,
    ) + b_ref[...]
    i_g = gates[:, :D]
    g_g = gates[:, 2 * D:3 * D]
    o_g = gates[:, 3 * D:]
    cc = jax.nn.sigmoid(i_g) * jnp.tanh(g_g)
    out_ref[...] = jax.nn.sigmoid(o_g) * jnp.tanh(cc)


_lstm_call = pl.pallas_call(
    _lstm_body,
    out_shape=jax.ShapeDtypeStruct((D, D), jnp.float32),
)

ROWB = 1024
NBLK = NPAD // ROWB


def _xws_body(x_ref, wn_ref, d0_ref, d1_ref, xws_ref, dis_ref):
    xw = jnp.dot(x_ref[...], wn_ref[...], preferred_element_type=jnp.float32)
    dis = lax.rsqrt(d0_ref[...] + d1_ref[...] + 1.0)
    dis_ref[...] = dis
    xws_ref[...] = dis * xw


_xws_call = pl.pallas_call(
    _xws_body,
    grid=(NBLK,),
    in_specs=[
        pl.BlockSpec((ROWB, D), lambda i: (i, 0)),
        pl.BlockSpec((D, D), lambda i: (0, 0)),
        pl.BlockSpec((ROWB, 1), lambda i: (i, 0)),
        pl.BlockSpec((ROWB, 1), lambda i: (i, 0)),
    ],
    out_specs=[
        pl.BlockSpec((ROWB, D), lambda i: (i, 0)),
        pl.BlockSpec((ROWB, 1), lambda i: (i, 0)),
    ],
    out_shape=[
        jax.ShapeDtypeStruct((NPAD, D), jnp.float32),
        jax.ShapeDtypeStruct((NPAD, 1), jnp.float32),
    ],
)


def _final_body(s0_ref, s1_ref, xws_ref, dis_ref, lw_ref, lb_ref, y_ref):
    h = jnp.maximum(
        dis_ref[...] * (s0_ref[...] + s1_ref[...] + xws_ref[...]), 0.0)
    y = jnp.dot(h, lw_ref[...], preferred_element_type=jnp.float32)
    y_ref[...] = y + lb_ref[0, 0]


_final_call = pl.pallas_call(
    _final_body,
    grid=(NBLK,),
    in_specs=[
        pl.BlockSpec((ROWB, D), lambda i: (i, 0)),
        pl.BlockSpec((ROWB, D), lambda i: (i, 0)),
        pl.BlockSpec((ROWB, D), lambda i: (i, 0)),
        pl.BlockSpec((ROWB, 1), lambda i: (i, 0)),
        pl.BlockSpec((D, 8), lambda i: (0, 0)),
        pl.BlockSpec((1, 1), lambda i: (0, 0)),
    ],
    out_specs=pl.BlockSpec((ROWB, 8), lambda i: (i, 0)),
    out_shape=jax.ShapeDtypeStruct((NPAD, 8), jnp.float32),
)


def kernel(x, edge_index, edge_weight, W, W_ih, W_hh, b_ih, b_hh, lin_w, lin_b):
    src = edge_index[0]
    dst = edge_index[1]
    pad = EPAD - E
    zi = jnp.arange(pad, dtype=jnp.int32)
    srcp = jnp.concatenate([src, zi])
    dstp = jnp.concatenate([dst, zi])
    ewp = jnp.concatenate([edge_weight, jnp.zeros((pad,), jnp.float32)])

    dst3 = dstp.reshape(NW, CH, IDXC)
    ew3 = ewp.reshape(NW, CH, IDXC)
    src4 = srcp.reshape(NW, PH, PCH, IDXC)
    dst4 = dstp.reshape(NW, PH, PCH, IDXC)
    ew4 = ewp.reshape(NW, PH, PCH, IDXC)

    deg = _deg_call(dst3, ew3)
    d0 = deg[0].reshape(NPAD, 1)
    d1 = deg[1].reshape(NPAD, 1)

    w_new = _lstm_call(W, W_ih, (b_ih + b_hh).reshape(1, 4 * D))

    xp = jnp.pad(x, ((0, NPAD - N), (0, 0)))
    xws, dis = _xws_call(xp, w_new, d0, d1)

    s_part = _msg_call(xws, src4, dst4, ew4)

    lwp = jnp.pad(lin_w.T, ((0, 0), (0, 7)))
    yp = _final_call(s_part[0], s_part[1], xws, dis, lwp, lin_b.reshape(1, 1))
    return yp[:N, 0]
